# baseline (speedup 1.0000x reference)
import functools

import jax
import jax.numpy as jnp
from jax.experimental import pallas as pl
from jax.experimental.pallas import tpu as pltpu


def _moe_fused_kernel(x_ref, gw_ref, gb_ref, w1_ref, b1_ref, w2cat_ref,
                      b2_ref, out_ref, h_ref, *, num_experts, hidden):
    x = x_ref[...]
    tb = x.shape[0]
    n_exp = num_experts

    lane = jax.lax.broadcasted_iota(jnp.int32, (tb, n_exp), 1)
    logits = (jnp.dot(x, gw_ref[...], preferred_element_type=jnp.float32)
              + gb_ref[...])

    m1 = jnp.max(logits, axis=-1, keepdims=True)
    i1 = jnp.min(jnp.where(logits == m1, lane, n_exp), axis=-1, keepdims=True)
    rest = jnp.where(lane == i1, -jnp.inf, logits)
    m2 = jnp.max(rest, axis=-1, keepdims=True)
    i2 = jnp.min(jnp.where(rest == m2, lane, n_exp), axis=-1, keepdims=True)

    e2 = jnp.exp(m2 - m1)
    c1 = 1.0 / (1.0 + e2)
    c2 = 1.0 - c1

    xb = x.astype(jnp.bfloat16)

    d_in = x.shape[1]
    for e in range(n_exp):
        he = (jnp.dot(xb, w1_ref[e * d_in:(e + 1) * d_in, :],
                      preferred_element_type=jnp.float32)
              + b1_ref[e])
        he = jnp.maximum(he, 0.0)
        ce = jnp.where(i1 == e, c1, 0.0) + jnp.where(i2 == e, c2, 0.0)
        h_ref[:, e * hidden:(e + 1) * hidden] = (he * ce).astype(jnp.bfloat16)

    cmat = jnp.where(lane == i1, c1, 0.0) + jnp.where(lane == i2, c2, 0.0)
    y = jnp.dot(h_ref[...], w2cat_ref[...], preferred_element_type=jnp.float32)
    y = y + jnp.dot(cmat, b2_ref[...], preferred_element_type=jnp.float32)
    out_ref[...] = y


def kernel(x, gate_w, gate_b, w1, b1, w2, b2):
    batch, d_in = x.shape
    num_experts, _, hidden = w1.shape
    d_out = w2.shape[2]

    if batch >= 2048:
        batch_tile = 1024
    else:
        batch_tile = max(8, ((batch + 7) // 8) * 8)
    n_tiles = pl.cdiv(batch, batch_tile)
    padded = n_tiles * batch_tile
    if padded != batch:
        x = jnp.pad(x, ((0, padded - batch), (0, 0)))

    x_c = x.astype(jnp.float32)
    gw = gate_w.astype(jnp.float32)
    gb = gate_b.reshape(1, num_experts).astype(jnp.float32)
    if hidden == d_out:
        wall = jnp.concatenate(
            [w1.reshape(num_experts * d_in, hidden),
             w2.reshape(num_experts * hidden, d_out)],
            axis=0).astype(jnp.bfloat16)
        w1cat = wall[:num_experts * d_in]
        w2cat = wall[num_experts * d_in:]
    else:
        w1cat = w1.reshape(num_experts * d_in, hidden).astype(jnp.bfloat16)
        w2cat = w2.reshape(num_experts * hidden, d_out).astype(jnp.bfloat16)
    b1_3 = b1.reshape(num_experts, 1, hidden).astype(jnp.float32)
    b2_2 = b2.astype(jnp.float32)

    body = functools.partial(_moe_fused_kernel, num_experts=num_experts,
                             hidden=hidden)

    flops = 2 * padded * (d_in * num_experts
                          + num_experts * (d_in * hidden + hidden * d_out))
    bytes_accessed = 4 * (padded * (d_in + d_out)
                          + num_experts * (d_in * hidden + hidden * d_out)
                          + d_in * num_experts
                          + num_experts * (1 + hidden + d_out))
    cost = pl.CostEstimate(flops=int(flops),
                           transcendentals=int(padded),
                           bytes_accessed=int(bytes_accessed))

    out = pl.pallas_call(
        body,
        out_shape=jax.ShapeDtypeStruct((padded, d_out), jnp.float32),
        grid=(n_tiles,),
        in_specs=[
            pl.BlockSpec((batch_tile, d_in), lambda i: (i, 0)),
            pl.BlockSpec((d_in, num_experts), lambda i: (0, 0)),
            pl.BlockSpec((1, num_experts), lambda i: (0, 0)),
            pl.BlockSpec((num_experts * d_in, hidden), lambda i: (0, 0)),
            pl.BlockSpec((num_experts, 1, hidden), lambda i: (0, 0, 0)),
            pl.BlockSpec((num_experts * hidden, d_out), lambda i: (0, 0)),
            pl.BlockSpec((num_experts, d_out), lambda i: (0, 0)),
        ],
        out_specs=pl.BlockSpec((batch_tile, d_out), lambda i: (i, 0)),
        scratch_shapes=[
            pltpu.VMEM((batch_tile, num_experts * hidden), jnp.bfloat16),
        ],
        compiler_params=pltpu.CompilerParams(
            dimension_semantics=("parallel",),
            vmem_limit_bytes=60 * 1024 * 1024),
        cost_estimate=cost,
    )(x_c, gw, gb, w1cat, b1_3, w2cat, b2_2)
    return out[:batch]


# speedup vs baseline: 1.0011x; 1.0011x over previous
import functools

import jax
import jax.numpy as jnp
from jax.experimental import pallas as pl
from jax.experimental.pallas import tpu as pltpu


def _moe_mxu_kernel(x_ref, gwp_ref, gbp_ref, w1_ref, b1_ref, w2_ref,
                    b2p_ref, out_ref, h_ref, *, num_experts, d_in, hidden,
                    d_out, batch_tile):
    n_exp = num_experts
    mc = batch_tile // 2
    nk1 = d_in // 256
    nk2 = (n_exp * hidden) // 256
    nh = d_out // 256
    rows = [slice(ci * mc, (ci + 1) * mc) for ci in range(2)]
    xs = [[x_ref[rows[ci], kt * 256:(kt + 1) * 256] for kt in range(nk1)]
          for ci in range(2)]

    for kt in range(nk1):
        pltpu.matmul_push_rhs(gwp_ref[kt * 256:(kt + 1) * 256, :],
                              staging_register=0, mxu_index=0)
        for ci in range(2):
            pltpu.matmul_acc_lhs(ci * 128, xs[ci][kt], mxu_index=0,
                                 load_staged_rhs=0 if ci == 0 else None)
    glog = [pltpu.matmul_pop(ci * 128, (mc, 256), jnp.float32, 0)
            for ci in range(2)]

    lane = jax.lax.broadcasted_iota(jnp.int32, (mc, 256), 1)
    cmat, i1s, i2s, c1s, c2s = [], [], [], [], []
    for ci in range(2):
        logits = jnp.where(lane < n_exp, glog[ci] + gbp_ref[...], -jnp.inf)
        m1 = jnp.max(logits, axis=-1, keepdims=True)
        i1 = jnp.min(jnp.where(logits == m1, lane, n_exp), axis=-1,
                     keepdims=True)
        rest = jnp.where(lane == i1, -jnp.inf, logits)
        m2 = jnp.max(rest, axis=-1, keepdims=True)
        i2 = jnp.min(jnp.where(rest == m2, lane, n_exp), axis=-1,
                     keepdims=True)
        e2 = jnp.exp(m2 - m1)
        c1 = 1.0 / (1.0 + e2)
        c2 = 1.0 - c1
        cmat.append(jnp.where(lane == i1, c1, 0.0)
                    + jnp.where(lane == i2, c2, 0.0))
        i1s.append(i1)
        i2s.append(i2)
        c1s.append(c1)
        c2s.append(c2)

    for e in range(n_exp):
        for kt in range(nk1):
            r0 = e * d_in + kt * 256
            for mxu in range(nh):
                pltpu.matmul_push_rhs(
                    w1_ref[r0:r0 + 256, mxu * 256:(mxu + 1) * 256],
                    staging_register=kt % 2, mxu_index=mxu)
            for ci in range(2):
                for mxu in range(nh):
                    pltpu.matmul_acc_lhs(
                        ci * 128, xs[ci][kt], mxu_index=mxu,
                        load_staged_rhs=kt % 2 if ci == 0 else None)
        for ci in range(2):
            parts = [pltpu.matmul_pop(ci * 128, (mc, 256), jnp.float32, mxu)
                     for mxu in range(nh)]
            he = jnp.concatenate(parts, axis=1) + b1_ref[e]
            he = jnp.maximum(he, 0.0)
            ce = (jnp.where(i1s[ci] == e, c1s[ci], 0.0)
                  + jnp.where(i2s[ci] == e, c2s[ci], 0.0))
            h_ref[rows[ci], e * hidden:(e + 1) * hidden] = he * ce

    for kt in range(nk2):
        for mxu in range(nh):
            pltpu.matmul_push_rhs(
                w2_ref[kt * 256:(kt + 1) * 256, mxu * 256:(mxu + 1) * 256],
                staging_register=kt % 2, mxu_index=mxu)
        for ci in range(2):
            lhs = h_ref[rows[ci], kt * 256:(kt + 1) * 256]
            for mxu in range(nh):
                pltpu.matmul_acc_lhs(
                    ci * 128, lhs, mxu_index=mxu,
                    load_staged_rhs=kt % 2 if ci == 0 else None)
    ys = []
    for ci in range(2):
        parts = [pltpu.matmul_pop(ci * 128, (mc, 256), jnp.float32, mxu)
                 for mxu in range(nh)]
        ys.append(jnp.concatenate(parts, axis=1))

    b2mxu = nh - 1
    yb2 = [[None] * nh for _ in range(2)]
    for half in range(nh):
        pltpu.matmul_push_rhs(b2p_ref[:, half * 256:(half + 1) * 256],
                              staging_register=0, mxu_index=b2mxu)
        for ci in range(2):
            pltpu.matmul_acc_lhs(ci * 128, cmat[ci], mxu_index=b2mxu,
                                 load_staged_rhs=0 if ci == 0 else None)
        for ci in range(2):
            yb2[ci][half] = pltpu.matmul_pop(ci * 128, (mc, 256),
                                             jnp.float32, b2mxu)
    for ci in range(2):
        out_ref[rows[ci], :] = ys[ci] + jnp.concatenate(yb2[ci], axis=1)


def _moe_dot_kernel(x_ref, gw_ref, gb_ref, w1_ref, b1_ref, w2cat_ref,
                    b2_ref, out_ref, h_ref, *, num_experts, hidden):
    x = x_ref[...]
    tb = x.shape[0]
    n_exp = num_experts

    lane = jax.lax.broadcasted_iota(jnp.int32, (tb, n_exp), 1)
    logits = (jnp.dot(x, gw_ref[...], preferred_element_type=jnp.float32)
              + gb_ref[...])
    m1 = jnp.max(logits, axis=-1, keepdims=True)
    i1 = jnp.min(jnp.where(logits == m1, lane, n_exp), axis=-1, keepdims=True)
    rest = jnp.where(lane == i1, -jnp.inf, logits)
    m2 = jnp.max(rest, axis=-1, keepdims=True)
    i2 = jnp.min(jnp.where(rest == m2, lane, n_exp), axis=-1, keepdims=True)
    e2 = jnp.exp(m2 - m1)
    c1 = 1.0 / (1.0 + e2)
    c2 = 1.0 - c1

    xb = x.astype(jnp.bfloat16)
    d_in = x.shape[1]
    for e in range(n_exp):
        he = (jnp.dot(xb, w1_ref[e * d_in:(e + 1) * d_in, :],
                      preferred_element_type=jnp.float32) + b1_ref[e])
        he = jnp.maximum(he, 0.0)
        ce = jnp.where(i1 == e, c1, 0.0) + jnp.where(i2 == e, c2, 0.0)
        h_ref[:, e * hidden:(e + 1) * hidden] = (he * ce).astype(jnp.bfloat16)

    cmat = jnp.where(lane == i1, c1, 0.0) + jnp.where(lane == i2, c2, 0.0)
    y = jnp.dot(h_ref[...], w2cat_ref[...], preferred_element_type=jnp.float32)
    y = y + jnp.dot(cmat, b2_ref[...], preferred_element_type=jnp.float32)
    out_ref[...] = y


def _common(batch, d_in):
    if batch >= 2048:
        batch_tile = 1024
    else:
        batch_tile = max(8, ((batch + 7) // 8) * 8)
    n_tiles = pl.cdiv(batch, batch_tile)
    return batch_tile, n_tiles, n_tiles * batch_tile


def kernel(x, gate_w, gate_b, w1, b1, w2, b2):
    batch, d_in = x.shape
    num_experts, _, hidden = w1.shape
    d_out = w2.shape[2]
    batch_tile, n_tiles, padded = _common(batch, d_in)
    if padded != batch:
        x = jnp.pad(x, ((0, padded - batch), (0, 0)))
    x_c = x.astype(jnp.float32)

    flops = 2 * padded * (d_in * num_experts
                          + num_experts * (d_in * hidden + hidden * d_out))
    bytes_accessed = 4 * (padded * (d_in + d_out)
                          + num_experts * (d_in * hidden + hidden * d_out)
                          + d_in * num_experts
                          + num_experts * (1 + hidden + d_out))
    cost = pl.CostEstimate(flops=int(flops),
                           transcendentals=int(padded),
                           bytes_accessed=int(bytes_accessed))

    explicit_ok = (d_in % 256 == 0 and hidden % 256 == 0
                   and d_out % 512 == 0 and batch_tile % 16 == 0
                   and num_experts <= 128 and batch_tile >= 16)

    if explicit_ok:
        gwp = jnp.pad(gate_w.astype(jnp.float32),
                      ((0, 0), (0, 256 - num_experts)))
        gbp = jnp.pad(gate_b.reshape(1, num_experts).astype(jnp.float32),
                      ((0, 0), (0, 256 - num_experts)))
        w1cat = w1.astype(jnp.float32).reshape(num_experts * d_in, hidden)
        b1_3 = b1.reshape(num_experts, 1, hidden).astype(jnp.float32)
        w2cat = w2.astype(jnp.float32).reshape(num_experts * hidden, d_out)
        b2p = jnp.pad(b2.astype(jnp.float32), ((0, 256 - num_experts), (0, 0)))

        body = functools.partial(_moe_mxu_kernel, num_experts=num_experts,
                                 d_in=d_in, hidden=hidden, d_out=d_out,
                                 batch_tile=batch_tile)
        out = pl.pallas_call(
            body,
            out_shape=jax.ShapeDtypeStruct((padded, d_out), jnp.float32),
            grid=(n_tiles,),
            in_specs=[
                pl.BlockSpec((batch_tile, d_in), lambda i: (i, 0)),
                pl.BlockSpec((d_in, 256), lambda i: (0, 0)),
                pl.BlockSpec((1, 256), lambda i: (0, 0)),
                pl.BlockSpec((num_experts * d_in, hidden), lambda i: (0, 0)),
                pl.BlockSpec((num_experts, 1, hidden), lambda i: (0, 0, 0)),
                pl.BlockSpec((num_experts * hidden, d_out), lambda i: (0, 0)),
                pl.BlockSpec((256, d_out), lambda i: (0, 0)),
            ],
            out_specs=pl.BlockSpec((batch_tile, d_out), lambda i: (i, 0)),
            scratch_shapes=[
                pltpu.VMEM((batch_tile, num_experts * hidden), jnp.float32),
            ],
            compiler_params=pltpu.CompilerParams(
                dimension_semantics=("parallel",),
                vmem_limit_bytes=60 * 1024 * 1024),
            cost_estimate=cost,
        )(x_c, gwp, gbp, w1cat, b1_3, w2cat, b2p)
        return out[:batch]

    gw = gate_w.astype(jnp.float32)
    gb = gate_b.reshape(1, num_experts).astype(jnp.float32)
    w1cat = w1.reshape(num_experts * d_in, hidden).astype(jnp.bfloat16)
    w2cat = w2.reshape(num_experts * hidden, d_out).astype(jnp.bfloat16)
    b1_3 = b1.reshape(num_experts, 1, hidden).astype(jnp.float32)
    b2_2 = b2.astype(jnp.float32)
    body = functools.partial(_moe_dot_kernel, num_experts=num_experts,
                             hidden=hidden)
    out = pl.pallas_call(
        body,
        out_shape=jax.ShapeDtypeStruct((padded, d_out), jnp.float32),
        grid=(n_tiles,),
        in_specs=[
            pl.BlockSpec((batch_tile, d_in), lambda i: (i, 0)),
            pl.BlockSpec((d_in, num_experts), lambda i: (0, 0)),
            pl.BlockSpec((1, num_experts), lambda i: (0, 0)),
            pl.BlockSpec((num_experts * d_in, hidden), lambda i: (0, 0)),
            pl.BlockSpec((num_experts, 1, hidden), lambda i: (0, 0, 0)),
            pl.BlockSpec((num_experts * hidden, d_out), lambda i: (0, 0)),
            pl.BlockSpec((num_experts, d_out), lambda i: (0, 0)),
        ],
        out_specs=pl.BlockSpec((batch_tile, d_out), lambda i: (i, 0)),
        scratch_shapes=[
            pltpu.VMEM((batch_tile, num_experts * hidden), jnp.bfloat16),
        ],
        compiler_params=pltpu.CompilerParams(
            dimension_semantics=("parallel",),
            vmem_limit_bytes=60 * 1024 * 1024),
        cost_estimate=cost,
    )(x_c, gw, gb, w1cat, b1_3, w2cat, b2_2)
    return out[:batch]


# speedup vs baseline: 1.0538x; 1.0526x over previous
import functools

import jax
import jax.numpy as jnp
from jax.experimental import pallas as pl
from jax.experimental.pallas import tpu as pltpu


def _moe_mxu_kernel(x_ref, gwp_ref, gbp_ref, w1_ref, b1_ref, w2_ref,
                    b2p_ref, out_ref, h_ref, *, num_experts, d_in, hidden,
                    d_out, batch_tile):
    n_exp = num_experts
    mc = batch_tile // 2
    nk1 = d_in // 256
    nk2 = (n_exp * hidden) // 256
    nh = d_out // 256
    rows = [slice(ci * mc, (ci + 1) * mc) for ci in range(2)]
    xs = [[x_ref[rows[ci], kt * 256:(kt + 1) * 256] for kt in range(nk1)]
          for ci in range(2)]

    gwp = jnp.pad(gwp_ref[...], ((0, 0), (0, 256 - n_exp)))
    for kt in range(nk1):
        pltpu.matmul_push_rhs(gwp[kt * 256:(kt + 1) * 256, :],
                              staging_register=0, mxu_index=0)
        for ci in range(2):
            pltpu.matmul_acc_lhs(ci * 128, xs[ci][kt], mxu_index=0,
                                 load_staged_rhs=0 if ci == 0 else None)
    glog = [pltpu.matmul_pop(ci * 128, (mc, 256), jnp.float32, 0)
            for ci in range(2)]

    lane = jax.lax.broadcasted_iota(jnp.int32, (mc, 256), 1)
    gbp = jnp.pad(gbp_ref[...], ((0, 0), (0, 256 - n_exp)))
    cmat, i1s, i2s, c1s, c2s = [], [], [], [], []
    for ci in range(2):
        logits = jnp.where(lane < n_exp, glog[ci] + gbp, -jnp.inf)
        m1 = jnp.max(logits, axis=-1, keepdims=True)
        i1 = jnp.min(jnp.where(logits == m1, lane, n_exp), axis=-1,
                     keepdims=True)
        rest = jnp.where(lane == i1, -jnp.inf, logits)
        m2 = jnp.max(rest, axis=-1, keepdims=True)
        i2 = jnp.min(jnp.where(rest == m2, lane, n_exp), axis=-1,
                     keepdims=True)
        e2 = jnp.exp(m2 - m1)
        c1 = 1.0 / (1.0 + e2)
        c2 = 1.0 - c1
        cmat.append(jnp.where(lane == i1, c1, 0.0)
                    + jnp.where(lane == i2, c2, 0.0))
        i1s.append(i1)
        i2s.append(i2)
        c1s.append(c1)
        c2s.append(c2)

    for e in range(n_exp):
        for kt in range(nk1):
            r0 = kt * 256
            for mxu in range(nh):
                pltpu.matmul_push_rhs(
                    w1_ref[e, r0:r0 + 256, mxu * 256:(mxu + 1) * 256],
                    staging_register=kt % 2, mxu_index=mxu)
            for ci in range(2):
                for mxu in range(nh):
                    pltpu.matmul_acc_lhs(
                        ci * 128, xs[ci][kt], mxu_index=mxu,
                        load_staged_rhs=kt % 2 if ci == 0 else None)
        for ci in range(2):
            parts = [pltpu.matmul_pop(ci * 128, (mc, 256), jnp.float32, mxu)
                     for mxu in range(nh)]
            he = jnp.concatenate(parts, axis=1) + b1_ref[e:e + 1, :]
            he = jnp.maximum(he, 0.0)
            ce = (jnp.where(i1s[ci] == e, c1s[ci], 0.0)
                  + jnp.where(i2s[ci] == e, c2s[ci], 0.0))
            h_ref[rows[ci], e * hidden:(e + 1) * hidden] = he * ce

    nsub = hidden // 256
    for kt in range(nk2):
        e, sub = kt // nsub, kt % nsub
        for mxu in range(nh):
            pltpu.matmul_push_rhs(
                w2_ref[e, sub * 256:(sub + 1) * 256,
                       mxu * 256:(mxu + 1) * 256],
                staging_register=kt % 2, mxu_index=mxu)
        for ci in range(2):
            lhs = h_ref[rows[ci], kt * 256:(kt + 1) * 256]
            for mxu in range(nh):
                pltpu.matmul_acc_lhs(
                    ci * 128, lhs, mxu_index=mxu,
                    load_staged_rhs=kt % 2 if ci == 0 else None)
    ys = []
    for ci in range(2):
        parts = [pltpu.matmul_pop(ci * 128, (mc, 256), jnp.float32, mxu)
                 for mxu in range(nh)]
        ys.append(jnp.concatenate(parts, axis=1))

    b2mxu = nh - 1
    yb2 = [[None] * nh for _ in range(2)]
    b2p = jnp.pad(b2p_ref[...], ((0, 256 - n_exp), (0, 0)))
    for half in range(nh):
        pltpu.matmul_push_rhs(b2p[:, half * 256:(half + 1) * 256],
                              staging_register=0, mxu_index=b2mxu)
        for ci in range(2):
            pltpu.matmul_acc_lhs(ci * 128, cmat[ci], mxu_index=b2mxu,
                                 load_staged_rhs=0 if ci == 0 else None)
        for ci in range(2):
            yb2[ci][half] = pltpu.matmul_pop(ci * 128, (mc, 256),
                                             jnp.float32, b2mxu)
    for ci in range(2):
        out_ref[rows[ci], :] = ys[ci] + jnp.concatenate(yb2[ci], axis=1)


def _moe_dot_kernel(x_ref, gw_ref, gb_ref, w1_ref, b1_ref, w2cat_ref,
                    b2_ref, out_ref, h_ref, *, num_experts, hidden):
    x = x_ref[...]
    tb = x.shape[0]
    n_exp = num_experts

    lane = jax.lax.broadcasted_iota(jnp.int32, (tb, n_exp), 1)
    logits = (jnp.dot(x, gw_ref[...], preferred_element_type=jnp.float32)
              + gb_ref[...])
    m1 = jnp.max(logits, axis=-1, keepdims=True)
    i1 = jnp.min(jnp.where(logits == m1, lane, n_exp), axis=-1, keepdims=True)
    rest = jnp.where(lane == i1, -jnp.inf, logits)
    m2 = jnp.max(rest, axis=-1, keepdims=True)
    i2 = jnp.min(jnp.where(rest == m2, lane, n_exp), axis=-1, keepdims=True)
    e2 = jnp.exp(m2 - m1)
    c1 = 1.0 / (1.0 + e2)
    c2 = 1.0 - c1

    xb = x.astype(jnp.bfloat16)
    d_in = x.shape[1]
    for e in range(n_exp):
        he = (jnp.dot(xb, w1_ref[e * d_in:(e + 1) * d_in, :],
                      preferred_element_type=jnp.float32) + b1_ref[e])
        he = jnp.maximum(he, 0.0)
        ce = jnp.where(i1 == e, c1, 0.0) + jnp.where(i2 == e, c2, 0.0)
        h_ref[:, e * hidden:(e + 1) * hidden] = (he * ce).astype(jnp.bfloat16)

    cmat = jnp.where(lane == i1, c1, 0.0) + jnp.where(lane == i2, c2, 0.0)
    y = jnp.dot(h_ref[...], w2cat_ref[...], preferred_element_type=jnp.float32)
    y = y + jnp.dot(cmat, b2_ref[...], preferred_element_type=jnp.float32)
    out_ref[...] = y


def _common(batch, d_in):
    if batch >= 2048:
        batch_tile = 1024
    else:
        batch_tile = max(8, ((batch + 7) // 8) * 8)
    n_tiles = pl.cdiv(batch, batch_tile)
    return batch_tile, n_tiles, n_tiles * batch_tile


def kernel(x, gate_w, gate_b, w1, b1, w2, b2):
    batch, d_in = x.shape
    num_experts, _, hidden = w1.shape
    d_out = w2.shape[2]
    batch_tile, n_tiles, padded = _common(batch, d_in)
    if padded != batch:
        x = jnp.pad(x, ((0, padded - batch), (0, 0)))
    x_c = x.astype(jnp.float32)

    flops = 2 * padded * (d_in * num_experts
                          + num_experts * (d_in * hidden + hidden * d_out))
    bytes_accessed = 4 * (padded * (d_in + d_out)
                          + num_experts * (d_in * hidden + hidden * d_out)
                          + d_in * num_experts
                          + num_experts * (1 + hidden + d_out))
    cost = pl.CostEstimate(flops=int(flops),
                           transcendentals=int(padded),
                           bytes_accessed=int(bytes_accessed))

    explicit_ok = (d_in % 256 == 0 and hidden % 256 == 0
                   and d_out % 512 == 0 and batch_tile % 16 == 0
                   and num_experts <= 128 and batch_tile >= 16)

    if explicit_ok:
        gw = gate_w.astype(jnp.float32)
        gb = gate_b.reshape(1, num_experts).astype(jnp.float32)
        w1_c = w1.astype(jnp.float32)
        b1_c = b1.astype(jnp.float32)
        w2_c = w2.astype(jnp.float32)
        b2_c = b2.astype(jnp.float32)

        body = functools.partial(_moe_mxu_kernel, num_experts=num_experts,
                                 d_in=d_in, hidden=hidden, d_out=d_out,
                                 batch_tile=batch_tile)
        out = pl.pallas_call(
            body,
            out_shape=jax.ShapeDtypeStruct((padded, d_out), jnp.float32),
            grid=(n_tiles,),
            in_specs=[
                pl.BlockSpec((batch_tile, d_in), lambda i: (i, 0)),
                pl.BlockSpec((d_in, num_experts), lambda i: (0, 0)),
                pl.BlockSpec((1, num_experts), lambda i: (0, 0)),
                pl.BlockSpec((num_experts, d_in, hidden),
                             lambda i: (0, 0, 0)),
                pl.BlockSpec((num_experts, hidden), lambda i: (0, 0)),
                pl.BlockSpec((num_experts, hidden, d_out),
                             lambda i: (0, 0, 0)),
                pl.BlockSpec((num_experts, d_out), lambda i: (0, 0)),
            ],
            out_specs=pl.BlockSpec((batch_tile, d_out), lambda i: (i, 0)),
            scratch_shapes=[
                pltpu.VMEM((batch_tile, num_experts * hidden), jnp.float32),
            ],
            compiler_params=pltpu.CompilerParams(
                dimension_semantics=("parallel",),
                vmem_limit_bytes=60 * 1024 * 1024),
            cost_estimate=cost,
        )(x_c, gw, gb, w1_c, b1_c, w2_c, b2_c)
        return out[:batch]

    gw = gate_w.astype(jnp.float32)
    gb = gate_b.reshape(1, num_experts).astype(jnp.float32)
    w1cat = w1.reshape(num_experts * d_in, hidden).astype(jnp.bfloat16)
    w2cat = w2.reshape(num_experts * hidden, d_out).astype(jnp.bfloat16)
    b1_3 = b1.reshape(num_experts, 1, hidden).astype(jnp.float32)
    b2_2 = b2.astype(jnp.float32)
    body = functools.partial(_moe_dot_kernel, num_experts=num_experts,
                             hidden=hidden)
    out = pl.pallas_call(
        body,
        out_shape=jax.ShapeDtypeStruct((padded, d_out), jnp.float32),
        grid=(n_tiles,),
        in_specs=[
            pl.BlockSpec((batch_tile, d_in), lambda i: (i, 0)),
            pl.BlockSpec((d_in, num_experts), lambda i: (0, 0)),
            pl.BlockSpec((1, num_experts), lambda i: (0, 0)),
            pl.BlockSpec((num_experts * d_in, hidden), lambda i: (0, 0)),
            pl.BlockSpec((num_experts, 1, hidden), lambda i: (0, 0, 0)),
            pl.BlockSpec((num_experts * hidden, d_out), lambda i: (0, 0)),
            pl.BlockSpec((num_experts, d_out), lambda i: (0, 0)),
        ],
        out_specs=pl.BlockSpec((batch_tile, d_out), lambda i: (i, 0)),
        scratch_shapes=[
            pltpu.VMEM((batch_tile, num_experts * hidden), jnp.bfloat16),
        ],
        compiler_params=pltpu.CompilerParams(
            dimension_semantics=("parallel",),
            vmem_limit_bytes=60 * 1024 * 1024),
        cost_estimate=cost,
    )(x_c, gw, gb, w1cat, b1_3, w2cat, b2_2)
    return out[:batch]
